# initial kernel scaffold (unmeasured)
import jax
import jax.numpy as jnp
from jax import lax
from jax.experimental import pallas as pl
from jax.experimental.pallas import tpu as pltpu

N_DEV = 8


def kernel(x, w_mat, scale_x, scale_w):
    m_per, k = x.shape
    n_per = w_mat.shape[1]
    scale = (scale_x * scale_w).reshape(1, 1).astype(jnp.float32)

    def body(x_ref, w_ref, s_ref, out_ref, gather_ref, send_sems, recv_sems):
        my = lax.axis_index("i")
        left = lax.rem(my + N_DEV - 1, N_DEV)
        right = lax.rem(my + 1, N_DEV)

        barrier_sem = pltpu.get_barrier_semaphore()
        for nbr in [left, right]:
            pl.semaphore_signal(
                barrier_sem, inc=1,
                device_id=(nbr,), device_id_type=pl.DeviceIdType.MESH,
            )
        pl.semaphore_wait(barrier_sem, 2)

        s = s_ref[0, 0]

        def compute_block(o):
            chunk = gather_ref[o].astype(jnp.bfloat16)
            w = w_ref[...].astype(jnp.bfloat16)
            acc = jnp.dot(chunk, w, preferred_element_type=jnp.float32)
            out_ref[pl.ds(o * m_per, m_per), :] = jnp.maximum(acc * s, 0.0)

        gather_ref[my] = x_ref[...]
        compute_block(my)

        for h in range(N_DEV - 1):
            s_idx = lax.rem(my + N_DEV - h, N_DEV)
            r_idx = lax.rem(my + N_DEV - h - 1, N_DEV)
            rdma = pltpu.make_async_remote_copy(
                src_ref=gather_ref.at[s_idx],
                dst_ref=gather_ref.at[s_idx],
                send_sem=send_sems.at[h],
                recv_sem=recv_sems.at[h],
                device_id=(right,),
                device_id_type=pl.DeviceIdType.MESH,
            )
            rdma.start()
            rdma.wait()
            compute_block(r_idx)

    return pl.pallas_call(
        body,
        out_shape=jax.ShapeDtypeStruct((N_DEV * m_per, n_per), jnp.float32),
        in_specs=[
            pl.BlockSpec(memory_space=pltpu.VMEM),
            pl.BlockSpec(memory_space=pltpu.VMEM),
            pl.BlockSpec(memory_space=pltpu.SMEM),
        ],
        out_specs=pl.BlockSpec(memory_space=pltpu.VMEM),
        scratch_shapes=[
            pltpu.VMEM((N_DEV, m_per, k), x.dtype),
            pltpu.SemaphoreType.DMA((N_DEV - 1,)),
            pltpu.SemaphoreType.DMA((N_DEV - 1,)),
        ],
        compiler_params=pltpu.CompilerParams(collective_id=0),
    )(x, w_mat, scale)


# baseline (device time: 250450 ns/iter reference)
import jax
import jax.numpy as jnp
from jax import lax
from jax.experimental import pallas as pl
from jax.experimental.pallas import tpu as pltpu

N_DEV = 8


def kernel(x, w_mat, scale_x, scale_w):
    m_per, k = x.shape
    n_per = w_mat.shape[1]
    scale = (scale_x * scale_w).reshape(1, 1).astype(jnp.float32)
    x = x.astype(jnp.float8_e5m2)
    w_mat = w_mat.astype(jnp.bfloat16)

    def body(x_ref, w_ref, s_ref, out_ref, gather_ref, send_sems, recv_sems):
        my = lax.axis_index("i")
        left = lax.rem(my + N_DEV - 1, N_DEV)
        right = lax.rem(my + 1, N_DEV)

        barrier_sem = pltpu.get_barrier_semaphore()
        for nbr in [left, right]:
            pl.semaphore_signal(
                barrier_sem, inc=1,
                device_id=(nbr,), device_id_type=pl.DeviceIdType.MESH,
            )
        pl.semaphore_wait(barrier_sem, 2)

        s = s_ref[0, 0]

        def compute_block(o):
            chunk = gather_ref[o].astype(jnp.bfloat16)
            acc = jnp.dot(chunk, w_ref[...], preferred_element_type=jnp.float32)
            out_ref[pl.ds(o * m_per, m_per), :] = jnp.maximum(acc * s, 0.0)

        gather_ref[my] = x_ref[...]
        compute_block(my)

        for h in range(N_DEV - 1):
            s_idx = lax.rem(my + N_DEV - h, N_DEV)
            r_idx = lax.rem(my + N_DEV - h - 1, N_DEV)
            rdma = pltpu.make_async_remote_copy(
                src_ref=gather_ref.at[s_idx],
                dst_ref=gather_ref.at[s_idx],
                send_sem=send_sems.at[h],
                recv_sem=recv_sems.at[h],
                device_id=(right,),
                device_id_type=pl.DeviceIdType.MESH,
            )
            rdma.start()
            rdma.wait()
            compute_block(r_idx)

    return pl.pallas_call(
        body,
        out_shape=jax.ShapeDtypeStruct((N_DEV * m_per, n_per), jnp.float32),
        in_specs=[
            pl.BlockSpec(memory_space=pltpu.VMEM),
            pl.BlockSpec(memory_space=pltpu.VMEM),
            pl.BlockSpec(memory_space=pltpu.SMEM),
        ],
        out_specs=pl.BlockSpec(memory_space=pltpu.VMEM),
        scratch_shapes=[
            pltpu.VMEM((N_DEV, m_per, k), x.dtype),
            pltpu.SemaphoreType.DMA((N_DEV - 1,)),
            pltpu.SemaphoreType.DMA((N_DEV - 1,)),
        ],
        compiler_params=pltpu.CompilerParams(
            collective_id=0,
            vmem_limit_bytes=100 * 1024 * 1024,
        ),
    )(x, w_mat, scale)


# device time: 141035 ns/iter; 1.7758x vs baseline; 1.7758x over previous
import jax
import jax.numpy as jnp
from jax import lax
from jax.experimental import pallas as pl
from jax.experimental.pallas import tpu as pltpu

N_DEV = 8


def kernel(x, w_mat, scale_x, scale_w):
    m_per, k = x.shape
    n_per = w_mat.shape[1]
    half = m_per // 2
    scale = (scale_x * scale_w).reshape(1, 1).astype(jnp.float32)
    x = x.astype(jnp.float8_e5m2)
    w_mat = w_mat.astype(jnp.bfloat16)

    def body(x_ref, w_ref, s_ref, out_ref,
             cw_buf, ccw_buf, cw_ssem, cw_rsem, ccw_ssem, ccw_rsem):
        my = lax.axis_index("i")
        left = lax.rem(my + N_DEV - 1, N_DEV)
        right = lax.rem(my + 1, N_DEV)

        barrier_sem = pltpu.get_barrier_semaphore()
        for nbr in [left, right]:
            pl.semaphore_signal(
                barrier_sem, inc=1,
                device_id=(nbr,), device_id_type=pl.DeviceIdType.MESH,
            )
        pl.semaphore_wait(barrier_sem, 2)

        s = s_ref[0, 0]

        def compute_half(buf, o, row_off):
            chunk = buf[o].astype(jnp.bfloat16)
            acc = jnp.dot(chunk, w_ref[...], preferred_element_type=jnp.float32)
            out_ref[pl.ds(o * m_per + row_off, half), :] = jnp.maximum(acc * s, 0.0)

        cw_buf[my] = x_ref[:half, :]
        ccw_buf[my] = x_ref[half:, :]

        rdmas = []
        for h in range(N_DEV - 1):
            cw_s = lax.rem(my + N_DEV - h, N_DEV)
            ccw_s = lax.rem(my + h, N_DEV)
            cw = pltpu.make_async_remote_copy(
                src_ref=cw_buf.at[cw_s], dst_ref=cw_buf.at[cw_s],
                send_sem=cw_ssem.at[h], recv_sem=cw_rsem.at[h],
                device_id=(right,), device_id_type=pl.DeviceIdType.MESH,
            )
            ccw = pltpu.make_async_remote_copy(
                src_ref=ccw_buf.at[ccw_s], dst_ref=ccw_buf.at[ccw_s],
                send_sem=ccw_ssem.at[h], recv_sem=ccw_rsem.at[h],
                device_id=(left,), device_id_type=pl.DeviceIdType.MESH,
            )
            cw.start()
            ccw.start()
            prev_cw = lax.rem(my + N_DEV - h, N_DEV)
            prev_ccw = lax.rem(my + h, N_DEV)
            compute_half(cw_buf, prev_cw, 0)
            compute_half(ccw_buf, prev_ccw, half)
            cw.wait()
            ccw.wait()
        compute_half(cw_buf, lax.rem(my + 1, N_DEV), 0)
        compute_half(ccw_buf, lax.rem(my + N_DEV - 1, N_DEV), half)

    return pl.pallas_call(
        body,
        out_shape=jax.ShapeDtypeStruct((N_DEV * m_per, n_per), jnp.float32),
        in_specs=[
            pl.BlockSpec(memory_space=pltpu.VMEM),
            pl.BlockSpec(memory_space=pltpu.VMEM),
            pl.BlockSpec(memory_space=pltpu.SMEM),
        ],
        out_specs=pl.BlockSpec(memory_space=pltpu.VMEM),
        scratch_shapes=[
            pltpu.VMEM((N_DEV, half, k), jnp.float8_e5m2),
            pltpu.VMEM((N_DEV, half, k), jnp.float8_e5m2),
            pltpu.SemaphoreType.DMA((N_DEV - 1,)),
            pltpu.SemaphoreType.DMA((N_DEV - 1,)),
            pltpu.SemaphoreType.DMA((N_DEV - 1,)),
            pltpu.SemaphoreType.DMA((N_DEV - 1,)),
        ],
        compiler_params=pltpu.CompilerParams(
            collective_id=0,
            vmem_limit_bytes=100 * 1024 * 1024,
        ),
    )(x, w_mat, scale)


# device time: 136669 ns/iter; 1.8325x vs baseline; 1.0319x over previous
import jax
import jax.numpy as jnp
from jax import lax
from jax.experimental import pallas as pl
from jax.experimental.pallas import tpu as pltpu

N_DEV = 8


def kernel(x, w_mat, scale_x, scale_w):
    m_per, k = x.shape
    n_per = w_mat.shape[1]
    half = m_per // 2
    scale = (scale_x * scale_w).reshape(1, 1).astype(jnp.float32)
    x = x.astype(jnp.float8_e5m2)
    w_mat = w_mat.astype(jnp.float8_e5m2)

    def body(x_ref, w_ref, s_ref, out_ref,
             cw_buf, ccw_buf, cw_ssem, cw_rsem, ccw_ssem, ccw_rsem):
        my = lax.axis_index("i")
        left = lax.rem(my + N_DEV - 1, N_DEV)
        right = lax.rem(my + 1, N_DEV)

        barrier_sem = pltpu.get_barrier_semaphore()
        for nbr in [left, right]:
            pl.semaphore_signal(
                barrier_sem, inc=1,
                device_id=(nbr,), device_id_type=pl.DeviceIdType.MESH,
            )
        pl.semaphore_wait(barrier_sem, 2)

        s = s_ref[0, 0]

        def compute_half(buf, o, row_off):
            acc = jnp.dot(buf[o], w_ref[...], preferred_element_type=jnp.float32)
            out_ref[pl.ds(o * m_per + row_off, half), :] = jnp.maximum(acc * s, 0.0)

        cw_buf[my] = x_ref[:half, :]
        ccw_buf[my] = x_ref[half:, :]

        rdmas = []
        for h in range(N_DEV - 1):
            cw_s = lax.rem(my + N_DEV - h, N_DEV)
            ccw_s = lax.rem(my + h, N_DEV)
            cw = pltpu.make_async_remote_copy(
                src_ref=cw_buf.at[cw_s], dst_ref=cw_buf.at[cw_s],
                send_sem=cw_ssem.at[h], recv_sem=cw_rsem.at[h],
                device_id=(right,), device_id_type=pl.DeviceIdType.MESH,
            )
            ccw = pltpu.make_async_remote_copy(
                src_ref=ccw_buf.at[ccw_s], dst_ref=ccw_buf.at[ccw_s],
                send_sem=ccw_ssem.at[h], recv_sem=ccw_rsem.at[h],
                device_id=(left,), device_id_type=pl.DeviceIdType.MESH,
            )
            cw.start()
            ccw.start()
            prev_cw = lax.rem(my + N_DEV - h, N_DEV)
            prev_ccw = lax.rem(my + h, N_DEV)
            compute_half(cw_buf, prev_cw, 0)
            compute_half(ccw_buf, prev_ccw, half)
            cw.wait()
            ccw.wait()
        compute_half(cw_buf, lax.rem(my + 1, N_DEV), 0)
        compute_half(ccw_buf, lax.rem(my + N_DEV - 1, N_DEV), half)

    return pl.pallas_call(
        body,
        out_shape=jax.ShapeDtypeStruct((N_DEV * m_per, n_per), jnp.float32),
        in_specs=[
            pl.BlockSpec(memory_space=pltpu.VMEM),
            pl.BlockSpec(memory_space=pltpu.VMEM),
            pl.BlockSpec(memory_space=pltpu.SMEM),
        ],
        out_specs=pl.BlockSpec(memory_space=pltpu.VMEM),
        scratch_shapes=[
            pltpu.VMEM((N_DEV, half, k), jnp.float8_e5m2),
            pltpu.VMEM((N_DEV, half, k), jnp.float8_e5m2),
            pltpu.SemaphoreType.DMA((N_DEV - 1,)),
            pltpu.SemaphoreType.DMA((N_DEV - 1,)),
            pltpu.SemaphoreType.DMA((N_DEV - 1,)),
            pltpu.SemaphoreType.DMA((N_DEV - 1,)),
        ],
        compiler_params=pltpu.CompilerParams(
            collective_id=0,
            vmem_limit_bytes=100 * 1024 * 1024,
        ),
    )(x, w_mat, scale)
